# Initial kernel scaffold; baseline (speedup 1.0000x reference)
import jax
import jax.numpy as jnp
from jax.experimental import pallas as pl


def kernel(x, edge_index, W1, b1, W2, b2, W3, b3):
    raise NotImplementedError("write your pallas kernel here")


# speedup vs baseline: 3.7678x; 3.7678x over previous
import functools

import jax
import jax.numpy as jnp
from jax import lax
from jax.experimental import pallas as pl
from jax.experimental.pallas import tpu as pltpu
from jax.experimental.pallas import tpu_sc as plsc

NCORE = 2
NSUB = 16
NW = NCORE * NSUB
CHUNK = 128


def _cdiv(a, b):
    return (a + b - 1) // b


def _make_agg(NA, F, EPW):
    CPW = EPW // CHUNK
    RPS = NA // NSUB
    mesh = plsc.VectorSubcoreMesh(core_axis_name="c", subcore_axis_name="s")

    @functools.partial(
        pl.kernel,
        out_type=jax.ShapeDtypeStruct((NCORE * NA, F), jnp.float32),
        mesh=mesh,
        scratch_types=[
            pltpu.VMEM_SHARED((NA, F), jnp.float32),
            pltpu.VMEM((CHUNK,), jnp.int32),
            pltpu.VMEM((CHUNK,), jnp.int32),
            pltpu.VMEM((CHUNK, F), jnp.float32),
            pltpu.SemaphoreType.DMA,
        ],
    )
    def agg(y_hbm, src_hbm, dst_hbm, zz_hbm, out_hbm, acc, sidx, didx, rows, sem):
        cid = lax.axis_index("c")
        sid = lax.axis_index("s")
        pltpu.sync_copy(zz_hbm.at[pl.ds(sid * RPS, RPS)],
                        acc.at[pl.ds(sid * RPS, RPS)])
        plsc.subcore_barrier()
        wid = cid * NSUB + sid

        @pl.loop(0, CPW)
        def _(k):
            base = wid * EPW + k * CHUNK
            pltpu.sync_copy(src_hbm.at[pl.ds(base, CHUNK)], sidx)
            pltpu.sync_copy(dst_hbm.at[pl.ds(base, CHUNK)], didx)
            pltpu.async_copy(y_hbm.at[sidx], rows, sem).wait()
            pltpu.sync_copy(rows, acc.at[didx], add=True)

        plsc.subcore_barrier()
        pltpu.sync_copy(acc.at[pl.ds(sid * RPS, RPS)],
                        out_hbm.at[pl.ds(cid * NA + sid * RPS, RPS)])

    return agg


def _pre_body(x_ref, w1b_ref, yaug_ref):
    x = x_ref[...]
    y = jnp.dot(x, w1b_ref[...], preferred_element_type=jnp.float32)
    cols = lax.broadcasted_iota(jnp.int32, (x.shape[0], 64), 1)
    extra = jnp.where(cols == 0, 1.0, 0.0).astype(jnp.float32)
    yaug_ref[...] = jnp.concatenate([y, extra], axis=1)


def _mid_body(n, a0_ref, a1_ref, x_ref, w1a_ref, b1_ref, w2a_ref, w2b_ref,
              z_ref, h1a_ref, rdeg_ref):
    s = a0_ref[0:n, 0:64] + a1_ref[0:n, 0:64]
    deg = a0_ref[0:n, 64:65] + a1_ref[0:n, 64:65]
    rdeg = 1.0 / jnp.maximum(deg, 1.0)
    xa = jnp.dot(x_ref[...], w1a_ref[...], preferred_element_type=jnp.float32)
    h1 = jnp.maximum(xa + s * rdeg + b1_ref[...], 0.0)
    z = jnp.dot(h1, w2b_ref[...], preferred_element_type=jnp.float32)
    z_ref[...] = jnp.pad(z, ((0, 0), (0, 96)))
    h1a_ref[...] = jnp.dot(h1, w2a_ref[...], preferred_element_type=jnp.float32)
    rdeg_ref[...] = rdeg


def _post_body(n, a0_ref, a1_ref, h1a_ref, rdeg_ref, b2_ref, w3_ref, b3_ref,
               out_ref):
    s2 = a0_ref[0:n, 0:32] + a1_ref[0:n, 0:32]
    mean2 = s2 * rdeg_ref[...]
    h2 = jnp.maximum(h1a_ref[...] + mean2 + b2_ref[...], 0.0)
    out_ref[...] = (jnp.dot(h2, w3_ref[...], preferred_element_type=jnp.float32)
                    + b3_ref[...])


def kernel(x, edge_index, W1, b1, W2, b2, W3, b3):
    N, D = x.shape
    E = edge_index.shape[1]
    F1 = W1.shape[1]
    F2 = W2.shape[1]

    EPW = _cdiv(E, NW * CHUNK) * CHUNK
    E_pad = EPW * NW
    NA = _cdiv(N + 1, NSUB * 8) * NSUB * 8
    trash = NA - N

    src = edge_index[0]
    dst = edge_index[1]
    pad_e = E_pad - E
    if pad_e:
        src = jnp.concatenate([src, jnp.zeros((pad_e,), jnp.int32)])
        dst = jnp.concatenate(
            [dst, N + (jnp.arange(pad_e, dtype=jnp.int32) % trash)])

    W1a, W1b = W1[:D], W1[D:]
    W2a, W2b = W2[:F1], W2[F1:]
    FA = 128
    zz = jnp.zeros((NA, FA), jnp.float32)

    yaug = pl.pallas_call(
        _pre_body,
        out_shape=jax.ShapeDtypeStruct((N, FA), jnp.float32),
    )(x, W1b)

    agg1 = _make_agg(NA, FA, EPW)
    r1 = agg1(yaug, src, dst, zz)
    a10, a11 = r1[:NA], r1[NA:]

    z, h1a, rdeg = pl.pallas_call(
        functools.partial(_mid_body, N),
        out_shape=(
            jax.ShapeDtypeStruct((N, FA), jnp.float32),
            jax.ShapeDtypeStruct((N, F2), jnp.float32),
            jax.ShapeDtypeStruct((N, 1), jnp.float32),
        ),
    )(a10, a11, x, W1a, b1.reshape(1, F1), W2a, W2b)

    agg2 = _make_agg(NA, FA, EPW)
    r2 = agg2(z, src, dst, zz)
    a20, a21 = r2[:NA], r2[NA:]

    out = pl.pallas_call(
        functools.partial(_post_body, N),
        out_shape=jax.ShapeDtypeStruct((N, 1), jnp.float32),
    )(a20, a21, h1a, rdeg, b2.reshape(1, F2), W3, b3.reshape(1, 1))

    return out
